# baseline (speedup 1.0000x reference)
import functools

import jax
import jax.numpy as jnp
from jax import lax
from jax.experimental import pallas as pl
from jax.experimental.pallas import tpu as pltpu
from jax.experimental.pallas import tpu_sc as plsc

N_OBJS = 1000000
D_OBJ = 64
N_ART = 10
D_ART = 32
BATCH = 16384

NC = 2
NS = 16
NW = NC * NS
B_PER_W = BATCH // NW
CHUNK = 128
N_CHUNKS = B_PER_W // CHUNK
L = 16

SPLIT = 262144
REST = N_OBJS - SPLIT
R_PER_W = SPLIT // NW
CCH = 256
N_CCH = R_PER_W // CCH

TBLK = 16384
TOFF = SPLIT // TBLK

_mesh = plsc.VectorSubcoreMesh(core_axis_name="c", subcore_axis_name="s")


def _transpose_fuse_kernel(ws_ref, wa_ref, o_ref):
    o_ref[:, 0:D_OBJ] = jnp.transpose(ws_ref[...])
    o_ref[:, D_OBJ:2 * D_OBJ] = jnp.transpose(wa_ref[...])


SBLK = BATCH // 8


def _select_art_kernel(ga_ref, gb_ref, ii_ref, ai_ref, wr_ref,
                       dens_ref, col_ref, art_ref):
    mask = (ii_ref[...].reshape(SBLK, 1) < SPLIT)
    sel_a = ga_ref[...]
    sel_b = gb_ref[...]
    dens_ref[...] = jnp.where(mask, sel_a[:, 0:D_OBJ], sel_b[:, 0:D_OBJ])
    col_ref[...] = jnp.where(mask, sel_a[:, D_OBJ:2 * D_OBJ],
                             sel_b[:, D_OBJ:2 * D_OBJ])
    ids = ai_ref[...].reshape(SBLK, 1)
    onehot = (ids == lax.broadcasted_iota(jnp.int32, (SBLK, N_ART), 1)
              ).astype(jnp.float32)
    art_ref[...] = jnp.dot(onehot, wr_ref[...],
                           preferred_element_type=jnp.float32,
                           precision=lax.Precision.HIGHEST)


@jax.jit
def _lookup(instance_id, articulation_id, W_shape, W_app, W_art):
    wst = W_shape.T
    wat = W_app.T
    ws3 = wst.reshape(8, 8, N_OBJS)
    wa3 = wat.reshape(8, 8, N_OBJS)

    @functools.partial(
        pl.kernel,
        out_type=jax.ShapeDtypeStruct((SPLIT, 2 * D_OBJ), jnp.float32),
        mesh=_mesh,
        compiler_params=pltpu.CompilerParams(needs_layout_passes=False),
        scratch_types=[
            pltpu.VMEM((2 * D_OBJ // 8, 8, CCH), jnp.float32),
            pltpu.VMEM((CCH, 129), jnp.float32),
            pltpu.SemaphoreType.DMA,
        ],
    )
    def _sc_transpose(ws_hbm, wa_hbm, out_hbm, in_v, out_v, sem):
        wid = lax.axis_index("s") * NC + lax.axis_index("c")
        row0 = wid * R_PER_W

        @pl.loop(0, N_CCH)
        def _(k):
            col0 = row0 + k * CCH
            for a in range(8):
                pltpu.async_copy(
                    ws_hbm.at[a, :, pl.ds(col0, CCH)], in_v.at[a], sem)
                pltpu.async_copy(
                    wa_hbm.at[a, :, pl.ds(col0, CCH)], in_v.at[8 + a], sem)
            for a in range(8):
                pltpu.make_async_copy(
                    ws_hbm.at[a, :, pl.ds(col0, CCH)], in_v.at[a], sem).wait()
                pltpu.make_async_copy(
                    wa_hbm.at[a, :, pl.ds(col0, CCH)], in_v.at[8 + a], sem).wait()

            @pl.loop(0, CCH, step=L)
            def _(g):
                l_vec = g + lax.iota(jnp.int32, L)
                for d in range(2 * D_OBJ):
                    vals = plsc.load_gather(
                        in_v, [jnp.full((L,), d >> 3, jnp.int32),
                               jnp.full((L,), d & 7, jnp.int32),
                               l_vec])
                    plsc.store_scatter(
                        out_v, [l_vec, jnp.full((L,), d, jnp.int32)], vals)

            pltpu.sync_copy(out_v.at[:, pl.ds(0, 2 * D_OBJ)],
                            out_hbm.at[pl.ds(col0, CCH)])

    table_sc = _sc_transpose(ws3, wa3)

    n_tblk = pl.cdiv(REST, TBLK)
    table_tc = pl.pallas_call(
        _transpose_fuse_kernel,
        grid=(n_tblk,),
        in_specs=[
            pl.BlockSpec((D_OBJ, TBLK), lambda i: (0, i + TOFF)),
            pl.BlockSpec((D_OBJ, TBLK), lambda i: (0, i + TOFF)),
        ],
        out_specs=pl.BlockSpec((TBLK, 2 * D_OBJ), lambda i: (i, 0)),
        out_shape=jax.ShapeDtypeStruct((REST, 2 * D_OBJ), jnp.float32),
    )(wst, wat)

    @functools.partial(
        pl.kernel,
        out_type=(
            jax.ShapeDtypeStruct((BATCH, 2 * D_OBJ), jnp.float32),
            jax.ShapeDtypeStruct((BATCH, 2 * D_OBJ), jnp.float32),
        ),
        mesh=_mesh,
        scratch_types=[
            pltpu.VMEM((B_PER_W,), jnp.int32),
            pltpu.VMEM((B_PER_W,), jnp.int32),
            pltpu.VMEM((B_PER_W // 2, 2 * D_OBJ), jnp.float32),
            pltpu.VMEM((B_PER_W // 2, 2 * D_OBJ), jnp.float32),
            pltpu.SemaphoreType.DMA,
        ],
    )
    def _gather(ta_hbm, tb_hbm, ii_hbm, outa_hbm, outb_hbm,
                ia_v, ib_v, rowsa_v, rowsb_v, sem):
        wid = lax.axis_index("s") * NC + lax.axis_index("c")
        base = wid * B_PER_W

        pltpu.sync_copy(ii_hbm.at[pl.ds(base, B_PER_W)], ia_v)

        @pl.loop(0, B_PER_W, step=L)
        def _(j0):
            ii = ia_v[pl.ds(j0, L)]
            in_a = ii < SPLIT
            ib_v[pl.ds(j0, L)] = jnp.where(in_a, ii >> 2, ii - SPLIT)
            ia_v[pl.ds(j0, L)] = jnp.where(in_a, ii, ii - SPLIT)

        for h in range(2):
            hb = h * (B_PER_W // 2)
            for t in range(N_CHUNKS // 2):
                cb = hb + t * CHUNK
                lb = t * CHUNK
                pltpu.async_copy(ta_hbm.at[ia_v.at[pl.ds(cb, CHUNK)]],
                                 rowsa_v.at[pl.ds(lb, CHUNK)], sem)
                pltpu.async_copy(tb_hbm.at[ib_v.at[pl.ds(cb, CHUNK)]],
                                 rowsb_v.at[pl.ds(lb, CHUNK)], sem)
            for t in range(N_CHUNKS // 2):
                cb = hb + t * CHUNK
                lb = t * CHUNK
                pltpu.make_async_copy(ta_hbm.at[ia_v.at[pl.ds(cb, CHUNK)]],
                                      rowsa_v.at[pl.ds(lb, CHUNK)], sem).wait()
                pltpu.make_async_copy(tb_hbm.at[ib_v.at[pl.ds(cb, CHUNK)]],
                                      rowsb_v.at[pl.ds(lb, CHUNK)], sem).wait()
            pltpu.sync_copy(rowsa_v, outa_hbm.at[pl.ds(base + hb, B_PER_W // 2)])
            pltpu.sync_copy(rowsb_v, outb_hbm.at[pl.ds(base + hb, B_PER_W // 2)])

    ga, gb = _gather(table_sc, table_tc, instance_id)

    density, color, articulation = pl.pallas_call(
        _select_art_kernel,
        grid=(8,),
        in_specs=[
            pl.BlockSpec((SBLK, 2 * D_OBJ), lambda i: (i, 0)),
            pl.BlockSpec((SBLK, 2 * D_OBJ), lambda i: (i, 0)),
            pl.BlockSpec((1, SBLK), lambda i: (0, i)),
            pl.BlockSpec((1, SBLK), lambda i: (0, i)),
            pl.BlockSpec((N_ART, D_ART), lambda i: (0, 0)),
        ],
        out_specs=(
            pl.BlockSpec((SBLK, D_OBJ), lambda i: (i, 0)),
            pl.BlockSpec((SBLK, D_OBJ), lambda i: (i, 0)),
            pl.BlockSpec((SBLK, D_ART), lambda i: (i, 0)),
        ),
        out_shape=(
            jax.ShapeDtypeStruct((BATCH, D_OBJ), jnp.float32),
            jax.ShapeDtypeStruct((BATCH, D_OBJ), jnp.float32),
            jax.ShapeDtypeStruct((BATCH, D_ART), jnp.float32),
        ),
    )(ga, gb, instance_id.reshape(1, BATCH),
      articulation_id.reshape(1, BATCH), W_art)

    return (density, color, articulation)


def kernel(instance_id, articulation_id, W_shape, W_app, W_art):
    return _lookup(
        instance_id.astype(jnp.int32),
        articulation_id.astype(jnp.int32),
        W_shape,
        W_app,
        W_art,
    )


# speedup vs baseline: 1.7888x; 1.7888x over previous
import functools

import jax
import jax.numpy as jnp
from jax import lax
from jax.experimental import pallas as pl
from jax.experimental.pallas import tpu as pltpu
from jax.experimental.pallas import tpu_sc as plsc

N_OBJS = 1000000
D_OBJ = 64
N_ART = 10
D_ART = 32
BATCH = 16384

NC = 2
NS = 16
NW = NC * NS
B_PER_W = BATCH // NW
CHUNK = 128
N_CHUNKS = B_PER_W // CHUNK

TBLK = 16384

_mesh = plsc.VectorSubcoreMesh(core_axis_name="c", subcore_axis_name="s")


def _transpose_fuse_kernel(ws_ref, wa_ref, o_ref):
    o_ref[:, 0:D_OBJ] = jnp.transpose(ws_ref[...])
    o_ref[:, D_OBJ:2 * D_OBJ] = jnp.transpose(wa_ref[...])


def _art_kernel(ai_ref, wr_ref, o_ref):
    ids = ai_ref[...].reshape(BATCH, 1)
    onehot = (ids == lax.broadcasted_iota(jnp.int32, (BATCH, N_ART), 1)
              ).astype(jnp.float32)
    o_ref[...] = jnp.dot(onehot, wr_ref[...],
                         preferred_element_type=jnp.float32,
                         precision=lax.Precision.HIGHEST)


@jax.jit
def _lookup(instance_id, articulation_id, W_shape, W_app, W_art):
    wst = W_shape.T
    wat = W_app.T

    n_tblk = pl.cdiv(N_OBJS, TBLK)
    fused = pl.pallas_call(
        _transpose_fuse_kernel,
        grid=(n_tblk,),
        in_specs=[
            pl.BlockSpec((D_OBJ, TBLK), lambda i: (0, i)),
            pl.BlockSpec((D_OBJ, TBLK), lambda i: (0, i)),
        ],
        out_specs=pl.BlockSpec((TBLK, 2 * D_OBJ), lambda i: (i, 0)),
        out_shape=jax.ShapeDtypeStruct((N_OBJS, 2 * D_OBJ), jnp.float32),
        compiler_params=pltpu.CompilerParams(
            dimension_semantics=("parallel",)),
    )(wst, wat)

    articulation = pl.pallas_call(
        _art_kernel,
        in_specs=[
            pl.BlockSpec((1, BATCH), lambda: (0, 0)),
            pl.BlockSpec((N_ART, D_ART), lambda: (0, 0)),
        ],
        out_specs=pl.BlockSpec((BATCH, D_ART), lambda: (0, 0)),
        out_shape=jax.ShapeDtypeStruct((BATCH, D_ART), jnp.float32),
    )(articulation_id.reshape(1, BATCH), W_art)

    @functools.partial(
        pl.kernel,
        out_type=jax.ShapeDtypeStruct((BATCH, 2 * D_OBJ), jnp.float32),
        mesh=_mesh,
        scratch_types=[
            pltpu.VMEM((B_PER_W,), jnp.int32),
            pltpu.VMEM((B_PER_W, 2 * D_OBJ), jnp.float32),
            pltpu.SemaphoreType.DMA,
            pltpu.SemaphoreType.DMA,
        ],
    )
    def _gather(tab_hbm, ii_hbm, out_hbm, ii_v, rows_v, sem_g, sem_o):
        wid = lax.axis_index("s") * NC + lax.axis_index("c")
        base = wid * B_PER_W

        pltpu.sync_copy(ii_hbm.at[pl.ds(base, B_PER_W)], ii_v)

        for t in range(N_CHUNKS):
            cb = t * CHUNK
            pltpu.async_copy(
                tab_hbm.at[ii_v.at[pl.ds(cb, CHUNK)]],
                rows_v.at[pl.ds(cb, CHUNK)], sem_g)
        for t in range(N_CHUNKS):
            cb = t * CHUNK
            pltpu.make_async_copy(
                tab_hbm.at[ii_v.at[pl.ds(cb, CHUNK)]],
                rows_v.at[pl.ds(cb, CHUNK)], sem_g).wait()

        pltpu.sync_copy(rows_v, out_hbm.at[pl.ds(base, B_PER_W)])

    fused_out = _gather(fused, instance_id)
    density = fused_out[:, 0:D_OBJ]
    color = fused_out[:, D_OBJ:2 * D_OBJ]
    return (density, color, articulation)


def kernel(instance_id, articulation_id, W_shape, W_app, W_art):
    return _lookup(
        instance_id.astype(jnp.int32),
        articulation_id.astype(jnp.int32),
        W_shape,
        W_app,
        W_art,
    )


# speedup vs baseline: 1.8032x; 1.0080x over previous
import functools

import jax
import jax.numpy as jnp
from jax import lax
from jax.experimental import pallas as pl
from jax.experimental.pallas import tpu as pltpu
from jax.experimental.pallas import tpu_sc as plsc

N_OBJS = 1000000
D_OBJ = 64
N_ART = 10
D_ART = 32
BATCH = 16384

NC = 2
NS = 16
NW = NC * NS
B_PER_W = BATCH // NW
CHUNK = 128
N_CHUNKS = B_PER_W // CHUNK

TBLK = 20480

_mesh = plsc.VectorSubcoreMesh(core_axis_name="c", subcore_axis_name="s")


def _transpose_fuse_kernel(ws_ref, wa_ref, o_ref):
    o_ref[:, 0:D_OBJ] = jnp.transpose(ws_ref[...])
    o_ref[:, D_OBJ:2 * D_OBJ] = jnp.transpose(wa_ref[...])


def _art_kernel(ai_ref, wr_ref, o_ref):
    ids = ai_ref[...].reshape(BATCH, 1)
    onehot = (ids == lax.broadcasted_iota(jnp.int32, (BATCH, N_ART), 1)
              ).astype(jnp.float32)
    o_ref[...] = jnp.dot(onehot, wr_ref[...],
                         preferred_element_type=jnp.float32,
                         precision=lax.Precision.HIGHEST)


@jax.jit
def _lookup(instance_id, articulation_id, W_shape, W_app, W_art):
    wst = W_shape.T
    wat = W_app.T

    n_tblk = pl.cdiv(N_OBJS, TBLK)
    fused = pl.pallas_call(
        _transpose_fuse_kernel,
        grid=(n_tblk,),
        in_specs=[
            pl.BlockSpec((D_OBJ, TBLK), lambda i: (0, i)),
            pl.BlockSpec((D_OBJ, TBLK), lambda i: (0, i)),
        ],
        out_specs=pl.BlockSpec((TBLK, 2 * D_OBJ), lambda i: (i, 0)),
        out_shape=jax.ShapeDtypeStruct((N_OBJS, 2 * D_OBJ), jnp.float32),
        compiler_params=pltpu.CompilerParams(
            dimension_semantics=("parallel",)),
    )(wst, wat)

    articulation = pl.pallas_call(
        _art_kernel,
        in_specs=[
            pl.BlockSpec((1, BATCH), lambda: (0, 0)),
            pl.BlockSpec((N_ART, D_ART), lambda: (0, 0)),
        ],
        out_specs=pl.BlockSpec((BATCH, D_ART), lambda: (0, 0)),
        out_shape=jax.ShapeDtypeStruct((BATCH, D_ART), jnp.float32),
    )(articulation_id.reshape(1, BATCH), W_art)

    @functools.partial(
        pl.kernel,
        out_type=jax.ShapeDtypeStruct((BATCH, 2 * D_OBJ), jnp.float32),
        mesh=_mesh,
        scratch_types=[
            pltpu.VMEM((B_PER_W,), jnp.int32),
            pltpu.VMEM((B_PER_W, 2 * D_OBJ), jnp.float32),
            pltpu.SemaphoreType.DMA,
            pltpu.SemaphoreType.DMA,
        ],
    )
    def _gather(tab_hbm, ii_hbm, out_hbm, ii_v, rows_v, sem_g, sem_o):
        wid = lax.axis_index("s") * NC + lax.axis_index("c")
        base = wid * B_PER_W

        pltpu.sync_copy(ii_hbm.at[pl.ds(base, B_PER_W)], ii_v)

        for t in range(N_CHUNKS):
            cb = t * CHUNK
            pltpu.async_copy(
                tab_hbm.at[ii_v.at[pl.ds(cb, CHUNK)]],
                rows_v.at[pl.ds(cb, CHUNK)], sem_g)
        for t in range(N_CHUNKS):
            cb = t * CHUNK
            pltpu.make_async_copy(
                tab_hbm.at[ii_v.at[pl.ds(cb, CHUNK)]],
                rows_v.at[pl.ds(cb, CHUNK)], sem_g).wait()

        pltpu.sync_copy(rows_v, out_hbm.at[pl.ds(base, B_PER_W)])

    fused_out = _gather(fused, instance_id)
    density = fused_out[:, 0:D_OBJ]
    color = fused_out[:, D_OBJ:2 * D_OBJ]
    return (density, color, articulation)


def kernel(instance_id, articulation_id, W_shape, W_app, W_art):
    return _lookup(
        instance_id.astype(jnp.int32),
        articulation_id.astype(jnp.int32),
        W_shape,
        W_app,
        W_art,
    )
